# Initial kernel scaffold; baseline (speedup 1.0000x reference)
import jax
import jax.numpy as jnp
from jax.experimental import pallas as pl


def kernel(fragment_embedding, fragment_cellxgene_ix, cell_n, gene_n, gene_ix, weight1, bias1):
    raise NotImplementedError("write your pallas kernel here")


# speedup vs baseline: 2.9878x; 2.9878x over previous
import jax
import jax.numpy as jnp
from jax import lax
from jax.experimental import pallas as pl
from jax.experimental.pallas import tpu as pltpu
from jax.experimental.pallas import tpu_sc as plsc

CELL_N = 4096
GENE_N = 256
N_FRAG = 1048576
D_EMB = 64
NSEG = CELL_N * GENE_N

FRAG_PER_ROW = 128
N_ROW = N_FRAG // FRAG_PER_ROW

F_BLK = 4096
R_BLK = F_BLK // FRAG_PER_ROW

NC = 2
NS = 16
ACC_N = NSEG // NC
TRASH = ACC_N
CHUNK_ROWS = 256
N_CHUNK = N_ROW // (NS * CHUNK_ROWS)
ACC_PER_TILE = ACC_N // NS
ZBUF = 2048


def _matvec(emb, w2, m_mask, g_sum):
    def body(x_ref, w_ref, m_ref, g_ref, o_ref):
        y = jnp.dot(x_ref[...], w_ref[...], preferred_element_type=jnp.float32)
        o_ref[...] = jnp.dot(g_ref[...], y * m_ref[...],
                             preferred_element_type=jnp.float32)

    return pl.pallas_call(
        body,
        grid=(N_FRAG // F_BLK,),
        in_specs=[
            pl.BlockSpec((F_BLK, D_EMB), lambda i: (i, 0)),
            pl.BlockSpec((D_EMB, FRAG_PER_ROW), lambda i: (0, 0)),
            pl.BlockSpec((F_BLK, FRAG_PER_ROW), lambda i: (0, 0)),
            pl.BlockSpec((R_BLK, F_BLK), lambda i: (0, 0)),
        ],
        out_specs=pl.BlockSpec((R_BLK, FRAG_PER_ROW), lambda i: (i, 0)),
        out_shape=jax.ShapeDtypeStruct((N_ROW, FRAG_PER_ROW), jnp.float32),
    )(emb, w2, m_mask, g_sum)


def _scatter_body(ids_hbm, vals_hbm, out_hbm, idx_v, val_v, zbuf, acc_sh):
    c = lax.axis_index("c")
    s = lax.axis_index("s")
    base = c * ACC_N

    def zfill(i, _):
        zbuf[pl.ds(i * 16, 16)] = jnp.zeros((16,), jnp.float32)
        return 0

    lax.fori_loop(0, ZBUF // 16, zfill, 0)

    def zcopy(k, _):
        pltpu.sync_copy(zbuf, acc_sh.at[pl.ds(s * ACC_PER_TILE + k * ZBUF, ZBUF)])
        return 0

    lax.fori_loop(0, ACC_PER_TILE // ZBUF, zcopy, 0)

    plsc.subcore_barrier()

    def chunk(t, _):
        row_base = (s * N_CHUNK + t) * CHUNK_ROWS
        pltpu.sync_copy(ids_hbm.at[pl.ds(row_base, CHUNK_ROWS)], idx_v)
        pltpu.sync_copy(vals_hbm.at[pl.ds(row_base, CHUNK_ROWS)], val_v)

        def xform(j, _):
            for k in range(FRAG_PER_ROW // 16):
                v = idx_v[j, pl.ds(k * 16, 16)] - base
                ok = (v >= 0) & (v < ACC_N)
                idx_v[j, pl.ds(k * 16, 16)] = jnp.where(
                    ok, v, jnp.full((16,), TRASH, jnp.int32))
            return 0

        lax.fori_loop(0, CHUNK_ROWS, xform, 0)

        def scat(j, _):
            pltpu.sync_copy(val_v.at[j], acc_sh.at[idx_v.at[j]], add=True)
            return 0

        lax.fori_loop(0, CHUNK_ROWS, scat, 0)
        return 0

    lax.fori_loop(0, N_CHUNK, chunk, 0)

    plsc.subcore_barrier()

    pltpu.sync_copy(acc_sh.at[pl.ds(s * ACC_PER_TILE, ACC_PER_TILE)],
                    out_hbm.at[pl.ds(base + s * ACC_PER_TILE, ACC_PER_TILE)])


def _scatter(ids2d, vals2d):
    mesh = plsc.VectorSubcoreMesh(core_axis_name="c", subcore_axis_name="s")
    return pl.kernel(
        _scatter_body,
        mesh=mesh,
        out_type=jax.ShapeDtypeStruct((NSEG,), jnp.float32),
        scratch_types=[
            pltpu.VMEM((CHUNK_ROWS, FRAG_PER_ROW), jnp.int32),
            pltpu.VMEM((CHUNK_ROWS, FRAG_PER_ROW), jnp.float32),
            pltpu.VMEM((ZBUF,), jnp.float32),
            pltpu.VMEM_SHARED((ACC_N + ZBUF,), jnp.float32),
        ],
    )(ids2d, vals2d)


def _finalize(acc128, bias_row):
    R = 256

    def body(p_ref, b_ref, o_ref):
        o_ref[...] = p_ref[...].reshape(R, GENE_N) + b_ref[...]

    return pl.pallas_call(
        body,
        grid=(CELL_N // R,),
        in_specs=[
            pl.BlockSpec((2 * R, FRAG_PER_ROW), lambda i: (i, 0)),
            pl.BlockSpec((1, GENE_N), lambda i: (0, 0)),
        ],
        out_specs=pl.BlockSpec((R, GENE_N), lambda i: (i, 0)),
        out_shape=jax.ShapeDtypeStruct((CELL_N, GENE_N), jnp.float32),
    )(acc128, bias_row)


def kernel(fragment_embedding, fragment_cellxgene_ix, cell_n, gene_n, gene_ix,
           weight1, bias1):
    offset = (cell_n * gene_n - NSEG).astype(jnp.int32)
    ids = fragment_cellxgene_ix.astype(jnp.int32) + offset
    ids = jnp.clip(ids, 0, NSEG - 1).reshape(N_ROW, FRAG_PER_ROW)

    w2 = weight1.astype(jnp.float32)[:, None] * jnp.ones(
        (1, FRAG_PER_ROW), jnp.float32)
    m_mask = jnp.tile(jnp.eye(FRAG_PER_ROW, dtype=jnp.float32), (R_BLK, 1))
    g_sum = jnp.repeat(jnp.eye(R_BLK, dtype=jnp.float32), FRAG_PER_ROW, axis=1)

    scalars = _matvec(fragment_embedding, w2, m_mask, g_sum)
    acc = _scatter(ids, scalars)

    bias_row = bias1[gene_ix].astype(jnp.float32).reshape(1, GENE_N)
    return _finalize(acc.reshape(N_ROW, FRAG_PER_ROW), bias_row)


# speedup vs baseline: 5.0503x; 1.6903x over previous
import jax
import jax.numpy as jnp
from jax import lax
from jax.experimental import pallas as pl
from jax.experimental.pallas import tpu as pltpu
from jax.experimental.pallas import tpu_sc as plsc

CELL_N = 4096
GENE_N = 256
N_FRAG = 1048576
D_EMB = 64
NSEG = CELL_N * GENE_N

FRAG_PER_ROW = 128
N_ROW = N_FRAG // FRAG_PER_ROW

F_BLK = 4096
R_BLK = F_BLK // FRAG_PER_ROW

NC = 2
NS = 16
NW = NC * NS
BIN_PER_W = NSEG // NW
TRASH = BIN_PER_W
CHUNK = 16384


def _matvec(emb, w2, m_mask, g_sum):
    def body(x_ref, w_ref, m_ref, g_ref, o_ref):
        y = jnp.dot(x_ref[...], w_ref[...], preferred_element_type=jnp.float32)
        o_ref[...] = jnp.dot(g_ref[...], y * m_ref[...],
                             preferred_element_type=jnp.float32)

    return pl.pallas_call(
        body,
        grid=(N_FRAG // F_BLK,),
        in_specs=[
            pl.BlockSpec((F_BLK, D_EMB), lambda i: (i, 0)),
            pl.BlockSpec((D_EMB, FRAG_PER_ROW), lambda i: (0, 0)),
            pl.BlockSpec((F_BLK, FRAG_PER_ROW), lambda i: (0, 0)),
            pl.BlockSpec((R_BLK, F_BLK), lambda i: (0, 0)),
        ],
        out_specs=pl.BlockSpec((R_BLK, FRAG_PER_ROW), lambda i: (i, 0)),
        out_shape=jax.ShapeDtypeStruct((N_ROW, FRAG_PER_ROW), jnp.float32),
    )(emb, w2, m_mask, g_sum)


def _scatter_body(ids_hbm, vals_hbm, bounds_hbm, out_hbm,
                  idx_v, val_v, bounds_v, acc):
    w = lax.axis_index("c") * NS + lax.axis_index("s")
    base = w * BIN_PER_W

    def zfill(i, _):
        acc[pl.ds(i * 16, 16)] = jnp.zeros((16,), jnp.float32)
        return 0

    lax.fori_loop(0, (BIN_PER_W + 16) // 16, zfill, 0)

    pltpu.sync_copy(bounds_hbm, bounds_v)
    bpair = bounds_v[pl.ds(w * 8, 16)]
    fs = bpair[0]
    fe = bpair[1]
    fs8 = pl.multiple_of(fs & ~7, 8)
    nch = (fe - fs8 + CHUNK - 1) // CHUNK

    def chunk_loop(i, _):
        off = pl.multiple_of(fs8 + i * CHUNK, 8)
        pltpu.sync_copy(ids_hbm.at[pl.ds(off, CHUNK)], idx_v)
        pltpu.sync_copy(vals_hbm.at[pl.ds(off, CHUNK)], val_v)

        def scat(j, _):
            v = idx_v[pl.ds(j * 16, 16)] - base
            ok = (v >= 0) & (v < BIN_PER_W)
            tgt = jnp.where(ok, v, jnp.full((16,), TRASH, jnp.int32))
            plsc.addupdate_scatter(acc, [tgt], val_v[pl.ds(j * 16, 16)])
            return 0

        lax.fori_loop(0, CHUNK // 16, scat, 0)
        return 0

    lax.fori_loop(0, nch, chunk_loop, 0)

    pltpu.sync_copy(acc.at[pl.ds(0, BIN_PER_W)],
                    out_hbm.at[pl.ds(base, BIN_PER_W)])


def _scatter(ids_p, vals_p, bounds):
    mesh = plsc.VectorSubcoreMesh(core_axis_name="c", subcore_axis_name="s")
    return pl.kernel(
        _scatter_body,
        mesh=mesh,
        out_type=jax.ShapeDtypeStruct((NSEG,), jnp.float32),
        compiler_params=pltpu.CompilerParams(needs_layout_passes=False),
        scratch_types=[
            pltpu.VMEM((CHUNK,), jnp.int32),
            pltpu.VMEM((CHUNK,), jnp.float32),
            pltpu.VMEM(((NW + 8) * 8,), jnp.int32),
            pltpu.VMEM((BIN_PER_W + 16,), jnp.float32),
        ],
    )(ids_p, vals_p, bounds)


def _finalize(acc128, bias_row):
    R = 256

    def body(p_ref, b_ref, o_ref):
        o_ref[...] = p_ref[...].reshape(R, GENE_N) + b_ref[...]

    return pl.pallas_call(
        body,
        grid=(CELL_N // R,),
        in_specs=[
            pl.BlockSpec((2 * R, FRAG_PER_ROW), lambda i: (i, 0)),
            pl.BlockSpec((1, GENE_N), lambda i: (0, 0)),
        ],
        out_specs=pl.BlockSpec((R, GENE_N), lambda i: (i, 0)),
        out_shape=jax.ShapeDtypeStruct((CELL_N, GENE_N), jnp.float32),
    )(acc128, bias_row)


def kernel(fragment_embedding, fragment_cellxgene_ix, cell_n, gene_n, gene_ix,
           weight1, bias1):
    offset = (cell_n * gene_n - NSEG).astype(jnp.int32)
    ids = fragment_cellxgene_ix.astype(jnp.int32) + offset
    ids = jnp.clip(ids, 0, NSEG - 1)

    edges = jnp.arange(NW + 1, dtype=jnp.int32) * BIN_PER_W
    b = jnp.searchsorted(ids, edges).astype(jnp.int32)
    bounds = jnp.pad(jnp.stack([b[:NW], b[1:NW + 1]], axis=1),
                     ((0, 8), (0, 6))).reshape(-1)
    ids_p = jnp.pad(ids, (0, CHUNK), constant_values=NSEG)

    w2 = weight1.astype(jnp.float32)[:, None] * jnp.ones(
        (1, FRAG_PER_ROW), jnp.float32)
    m_mask = jnp.tile(jnp.eye(FRAG_PER_ROW, dtype=jnp.float32), (R_BLK, 1))
    g_sum = jnp.repeat(jnp.eye(R_BLK, dtype=jnp.float32), FRAG_PER_ROW, axis=1)

    scalars = _matvec(fragment_embedding, w2, m_mask, g_sum)
    vals_p = jnp.pad(scalars.reshape(N_FRAG), (0, CHUNK))
    acc = _scatter(ids_p, vals_p, bounds)

    bias_row = bias1[gene_ix].astype(jnp.float32).reshape(1, GENE_N)
    return _finalize(acc.reshape(N_ROW, FRAG_PER_ROW), bias_row)
